# baseline (speedup 1.0000x reference)
import functools

import jax
import jax.numpy as jnp
from jax import lax
from jax.experimental import pallas as pl
from jax.experimental.pallas import tpu as pltpu
from jax.experimental.pallas import tpu_sc as plsc

N_IN = 2048
N_TOTAL = 8192
C = 128
K = 4


_QB = 256


def _topk_body(qx_ref, qy_ref, kx_ref, ky_ref, idx_ref):
    dx = qx_ref[...] - kx_ref[...]
    dy = qy_ref[...] - ky_ref[...]
    d = jnp.sqrt(dx * dx + dy * dy)
    iota_f = lax.broadcasted_iota(jnp.int32, (_QB, N_IN), 1).astype(jnp.float32)
    cols = []
    for _ in range(K):
        m = jnp.min(d, axis=1, keepdims=True)
        j = jnp.min(jnp.where(d == m, iota_f, jnp.float32(N_IN)),
                    axis=1, keepdims=True)
        d = jnp.where(iota_f == j, jnp.float32(jnp.inf), d)
        cols.append(j)
    idx_ref[...] = jnp.concatenate(cols, axis=1).astype(jnp.int32)


def _tc_topk(qx, qy, kx, ky):
    grid = N_TOTAL // _QB
    return pl.pallas_call(
        _topk_body,
        grid=(grid,),
        in_specs=[
            pl.BlockSpec((_QB, 1), lambda i: (i, 0)),
            pl.BlockSpec((_QB, 1), lambda i: (i, 0)),
            pl.BlockSpec((1, N_IN), lambda i: (0, 0)),
            pl.BlockSpec((1, N_IN), lambda i: (0, 0)),
        ],
        out_specs=pl.BlockSpec((_QB, K), lambda i: (i, 0)),
        out_shape=jax.ShapeDtypeStruct((N_TOTAL, K), jnp.int32),
    )(qx, qy, kx, ky)


_NC = 2
_NS = 16
_NW = _NC * _NS
_QPW = N_TOTAL // _NW
_QCHUNK = 32
_NCHUNK = _QPW // _QCHUNK


_CPW = C // _NW
_NSTEP = N_TOTAL // 16


_CSTEP = 128
_NCHK = _NSTEP // _CSTEP


def _gather_mean_body(values_hbm, idx_hbm, out_hbm,
                      idx_v, vr0, vr1, vr2, vr3, out_v):
    c = lax.axis_index("c")
    s = lax.axis_index("s")
    wid = s * _NC + c
    c0 = wid * _CPW
    vrows = (vr0, vr1, vr2, vr3)

    for r in range(_CPW):
        pltpu.sync_copy(values_hbm.at[c0 + r], vrows[r])

    quarter = jnp.float32(1.0 / K)

    for ch in range(_NCHK):
        pltpu.sync_copy(idx_hbm.at[:, pl.ds(ch * _CSTEP, _CSTEP)], idx_v)

        def step(st, carry):
            ihi = [lax.shift_right_logical(idx_v[k, st], 4) for k in range(K)]
            ilo = [lax.bitwise_and(idx_v[k, st], 15) for k in range(K)]
            for r in range(_CPW):
                acc = plsc.load_gather(vrows[r], [ihi[0], ilo[0]])
                for k in range(1, K):
                    acc = acc + plsc.load_gather(vrows[r], [ihi[k], ilo[k]])
                out_v[r, st] = acc * quarter
            return carry

        lax.fori_loop(0, _CSTEP, step, 0, unroll=4)

        for r in range(_CPW):
            pltpu.sync_copy(out_v.at[r],
                            out_hbm.at[c0 + r, pl.ds(ch * _CSTEP, _CSTEP)])


def _sc_gather_mean(values, idx_t):
    mesh = plsc.VectorSubcoreMesh(core_axis_name="c", subcore_axis_name="s")
    kern = pl.kernel(
        _gather_mean_body,
        out_type=jax.ShapeDtypeStruct((C, _NSTEP, 16), jnp.float32),
        mesh=mesh,
        compiler_params=pltpu.CompilerParams(
            needs_layout_passes=False, use_tc_tiling_on_sc=False),
        scratch_types=[
            pltpu.VMEM((K, _CSTEP, 16), jnp.int32),
            pltpu.VMEM((N_IN // 16, 16), jnp.float32),
            pltpu.VMEM((N_IN // 16, 16), jnp.float32),
            pltpu.VMEM((N_IN // 16, 16), jnp.float32),
            pltpu.VMEM((N_IN // 16, 16), jnp.float32),
            pltpu.VMEM((_CPW, _CSTEP, 16), jnp.float32),
        ],
    )
    return kern(values, idx_t)


def kernel(values, coords, new_coords, shift):
    all_coords = jnp.concatenate([coords, new_coords], axis=0)
    q = all_coords - shift
    qx = q[:, 0:1]
    qy = q[:, 1:2]
    kx = coords[:, 0][None, :]
    ky = coords[:, 1][None, :]
    idx = _tc_topk(qx, qy, kx, ky)
    idx_t = idx.T.reshape(K, _NSTEP, 16)
    out = _sc_gather_mean(values.reshape(C, N_IN // 16, 16), idx_t)
    return out.reshape(C, N_TOTAL)


# speedup vs baseline: 1.2658x; 1.2658x over previous
import functools

import jax
import jax.numpy as jnp
from jax import lax
from jax.experimental import pallas as pl
from jax.experimental.pallas import tpu as pltpu
from jax.experimental.pallas import tpu_sc as plsc

N_IN = 2048
N_TOTAL = 8192
C = 128
K = 4


_QB = 256
_NB = N_TOTAL // _QB


def _topk_body(qx_ref, qy_ref, coords_ref, idx_ref):
    kx = coords_ref[:, 0:1]
    ky = coords_ref[:, 1:2]
    dx = kx - qx_ref[...]
    dy = ky - qy_ref[...]
    d = jnp.sqrt(dx * dx + dy * dy)
    iota_f = lax.broadcasted_iota(jnp.int32, (N_IN, _QB), 0).astype(jnp.float32)
    cols = []
    for _ in range(K):
        m = jnp.min(d, axis=0, keepdims=True)
        j = jnp.min(jnp.where(d == m, iota_f, jnp.float32(N_IN)),
                    axis=0, keepdims=True)
        d = jnp.where(iota_f == j, jnp.float32(jnp.inf), d)
        cols.append(j)
    idx_ref[...] = jnp.concatenate(cols, axis=0)[None].astype(jnp.int32)


def _tc_topk(qx, qy, coords):
    return pl.pallas_call(
        _topk_body,
        grid=(_NB,),
        in_specs=[
            pl.BlockSpec((1, _QB), lambda i: (0, i)),
            pl.BlockSpec((1, _QB), lambda i: (0, i)),
            pl.BlockSpec((N_IN, 2), lambda i: (0, 0)),
        ],
        out_specs=pl.BlockSpec((1, K, _QB), lambda i: (i, 0, 0)),
        out_shape=jax.ShapeDtypeStruct((_NB, K, _QB), jnp.int32),
    )(qx, qy, coords)


_NC = 2
_NS = 16
_NW = _NC * _NS
_QPW = N_TOTAL // _NW
_QCHUNK = 32
_NCHUNK = _QPW // _QCHUNK


def _gather_mean_body(table_hbm, idx_hbm, out_hbm,
                      idx_v0, idx_v1, rows_v0, rows_v1, out_v, sem0, sem1):
    c = lax.axis_index("c")
    s = lax.axis_index("s")
    wid = s * _NC + c
    base_q = wid * _QPW
    idx_bufs = (idx_v0, idx_v1)
    row_bufs = (rows_v0, rows_v1)
    sems = (sem0, sem1)

    def start(ch):
        p = ch % 2
        for g in range(K):
            pltpu.sync_copy(idx_hbm.at[wid, g, pl.ds(ch * _QCHUNK, _QCHUNK)],
                            idx_bufs[p].at[pl.ds(g * _QCHUNK, _QCHUNK)])
        return pltpu.async_copy(table_hbm.at[idx_bufs[p]], row_bufs[p], sems[p])

    copies = [start(0)]
    for ch in range(_NCHUNK):
        if ch + 1 < _NCHUNK:
            copies.append(start(ch + 1))
        copies[ch].wait()
        rows_v = row_bufs[ch % 2]
        qb = base_q + ch * _QCHUNK

        def q_body(q, carry2):
            for l in range(C // 16):
                sl = pl.ds(l * 16, 16)
                acc = (rows_v[q, sl]
                       + rows_v[_QCHUNK + q, sl]
                       + rows_v[2 * _QCHUNK + q, sl]
                       + rows_v[3 * _QCHUNK + q, sl])
                out_v[q, sl] = acc * jnp.float32(1.0 / K)
            return carry2

        lax.fori_loop(0, _QCHUNK, q_body, 0, unroll=4)
        pltpu.sync_copy(out_v, out_hbm.at[pl.ds(qb, _QCHUNK)])


def _sc_gather_mean(table, idx):
    mesh = plsc.VectorSubcoreMesh(core_axis_name="c", subcore_axis_name="s")
    kern = pl.kernel(
        _gather_mean_body,
        out_type=jax.ShapeDtypeStruct((N_TOTAL, C), jnp.float32),
        mesh=mesh,
        scratch_types=[
            pltpu.VMEM((_QCHUNK * K,), jnp.int32),
            pltpu.VMEM((_QCHUNK * K,), jnp.int32),
            pltpu.VMEM((_QCHUNK * K, C), jnp.float32),
            pltpu.VMEM((_QCHUNK * K, C), jnp.float32),
            pltpu.VMEM((_QCHUNK, C), jnp.float32),
            pltpu.SemaphoreType.DMA,
            pltpu.SemaphoreType.DMA,
        ],
    )
    return kern(table, idx)


def kernel(values, coords, new_coords, shift):
    q2 = (jnp.concatenate([coords, new_coords], axis=0) - shift).T
    qx = q2[0:1]
    qy = q2[1:2]
    idx = _tc_topk(qx, qy, coords)
    table = values.T
    out_rows = _sc_gather_mean(table, idx)
    return out_rows.T


# speedup vs baseline: 1.4512x; 1.1465x over previous
import functools

import jax
import jax.numpy as jnp
from jax import lax
from jax.experimental import pallas as pl
from jax.experimental.pallas import tpu as pltpu
from jax.experimental.pallas import tpu_sc as plsc

N_IN = 2048
N_TOTAL = 8192
C = 128
K = 4


_QB = 256


def _topk_body(q_ref, sh_ref, kx_ref, ky_ref, idx_ref):
    qx = q_ref[:, 0:1] - sh_ref[0:1, 0:1]
    qy = q_ref[:, 1:2] - sh_ref[0:1, 1:2]
    dx = qx - kx_ref[...]
    dy = qy - ky_ref[...]
    d = jnp.sqrt(dx * dx + dy * dy)
    iota_f = lax.broadcasted_iota(jnp.int32, (_QB, N_IN), 1).astype(jnp.float32)
    cols = []
    for _ in range(K):
        m = jnp.min(d, axis=1, keepdims=True)
        j = jnp.min(jnp.where(d == m, iota_f, jnp.float32(N_IN)),
                    axis=1, keepdims=True)
        d = jnp.where(iota_f == j, jnp.float32(jnp.inf), d)
        cols.append(j)
    idx_ref[...] = jnp.concatenate(cols, axis=1).astype(jnp.int32)


def _tc_topk(all_coords, shift2d, kx, ky):
    grid = N_TOTAL // _QB
    return pl.pallas_call(
        _topk_body,
        grid=(grid,),
        in_specs=[
            pl.BlockSpec((_QB, 2), lambda i: (i, 0)),
            pl.BlockSpec((1, 2), lambda i: (0, 0)),
            pl.BlockSpec((1, N_IN), lambda i: (0, 0)),
            pl.BlockSpec((1, N_IN), lambda i: (0, 0)),
        ],
        out_specs=pl.BlockSpec((_QB, K), lambda i: (i, 0)),
        out_shape=jax.ShapeDtypeStruct((N_TOTAL, K), jnp.int32),
    )(all_coords, shift2d, kx, ky)


_NC = 2
_NS = 16
_NW = _NC * _NS
_QPW = N_TOTAL // _NW
_QCHUNK = 32
_NCHUNK = _QPW // _QCHUNK


def _gather_mean_body(table_hbm, idx_hbm, out_hbm,
                      idx_v0, idx_v1, rows_v0, rows_v1, out_v, sem0, sem1):
    c = lax.axis_index("c")
    s = lax.axis_index("s")
    wid = s * _NC + c
    base_q = wid * _QPW
    idx_bufs = (idx_v0, idx_v1)
    row_bufs = (rows_v0, rows_v1)
    sems = (sem0, sem1)

    def start(ch):
        qb = base_q + ch * _QCHUNK
        p = ch % 2
        pltpu.sync_copy(idx_hbm.at[pl.ds(qb * K, _QCHUNK * K)], idx_bufs[p])
        return pltpu.async_copy(table_hbm.at[idx_bufs[p]], row_bufs[p], sems[p])

    copies = [start(0)]
    for ch in range(_NCHUNK):
        if ch + 1 < _NCHUNK:
            copies.append(start(ch + 1))
        copies[ch].wait()
        rows_v = row_bufs[ch % 2]
        qb = base_q + ch * _QCHUNK

        def q_body(q, carry2):
            for l in range(C // 16):
                sl = pl.ds(l * 16, 16)
                acc = (rows_v[K * q, sl] + rows_v[K * q + 1, sl]
                       + rows_v[K * q + 2, sl] + rows_v[K * q + 3, sl])
                out_v[q, sl] = acc * jnp.float32(1.0 / K)
            return carry2

        lax.fori_loop(0, _QCHUNK, q_body, 0, unroll=4)
        pltpu.sync_copy(out_v, out_hbm.at[pl.ds(qb, _QCHUNK)])


def _sc_gather_mean(table, idx_flat):
    mesh = plsc.VectorSubcoreMesh(core_axis_name="c", subcore_axis_name="s")
    kern = pl.kernel(
        _gather_mean_body,
        out_type=jax.ShapeDtypeStruct((N_TOTAL, C), jnp.float32),
        mesh=mesh,
        scratch_types=[
            pltpu.VMEM((_QCHUNK * K,), jnp.int32),
            pltpu.VMEM((_QCHUNK * K,), jnp.int32),
            pltpu.VMEM((_QCHUNK * K, C), jnp.float32),
            pltpu.VMEM((_QCHUNK * K, C), jnp.float32),
            pltpu.VMEM((_QCHUNK, C), jnp.float32),
            pltpu.SemaphoreType.DMA,
            pltpu.SemaphoreType.DMA,
        ],
    )
    return kern(table, idx_flat)


def kernel(values, coords, new_coords, shift):
    all_coords = jnp.concatenate([coords, new_coords], axis=0)
    kx = coords[:, 0][None, :]
    ky = coords[:, 1][None, :]
    idx = _tc_topk(all_coords, shift[None, :], kx, ky)
    table = values.T
    out_rows = _sc_gather_mean(table, idx.reshape(-1))
    return out_rows.T


# speedup vs baseline: 1.5089x; 1.0397x over previous
import functools

import jax
import jax.numpy as jnp
from jax import lax
from jax.experimental import pallas as pl
from jax.experimental.pallas import tpu as pltpu
from jax.experimental.pallas import tpu_sc as plsc

N_IN = 2048
N_TOTAL = 8192
C = 128
K = 4


_QB = 512


def _topk_body(q_ref, sh_ref, kx_ref, ky_ref, idx_ref):
    qx = q_ref[:, 0:1] - sh_ref[0:1, 0:1]
    qy = q_ref[:, 1:2] - sh_ref[0:1, 1:2]
    dx = qx - kx_ref[...]
    dy = qy - ky_ref[...]
    d = jnp.sqrt(dx * dx + dy * dy)
    iota_f = lax.broadcasted_iota(jnp.int32, (_QB, N_IN), 1).astype(jnp.float32)
    cols = []
    for _ in range(K):
        m = jnp.min(d, axis=1, keepdims=True)
        j = jnp.min(jnp.where(d == m, iota_f, jnp.float32(N_IN)),
                    axis=1, keepdims=True)
        d = jnp.where(iota_f == j, jnp.float32(jnp.inf), d)
        cols.append(j)
    idx_ref[...] = jnp.concatenate(cols, axis=1).astype(jnp.int32)


def _tc_topk(all_coords, shift2d, kx, ky):
    grid = N_TOTAL // _QB
    return pl.pallas_call(
        _topk_body,
        grid=(grid,),
        in_specs=[
            pl.BlockSpec((_QB, 2), lambda i: (i, 0)),
            pl.BlockSpec((1, 2), lambda i: (0, 0)),
            pl.BlockSpec((1, N_IN), lambda i: (0, 0)),
            pl.BlockSpec((1, N_IN), lambda i: (0, 0)),
        ],
        out_specs=pl.BlockSpec((_QB, K), lambda i: (i, 0)),
        out_shape=jax.ShapeDtypeStruct((N_TOTAL, K), jnp.int32),
    )(all_coords, shift2d, kx, ky)


_NC = 2
_NS = 16
_NW = _NC * _NS
_QPW = N_TOTAL // _NW
_QCHUNK = 32
_NCHUNK = _QPW // _QCHUNK


_GDEPTH = 3


def _gather_mean_body(table_hbm, idx_hbm, out_hbm,
                      idx_all, rows_v0, rows_v1, rows_v2,
                      out_v0, out_v1,
                      gsem0, gsem1, gsem2, wsem0, wsem1):
    c = lax.axis_index("c")
    s = lax.axis_index("s")
    wid = s * _NC + c
    base_q = wid * _QPW
    row_bufs = (rows_v0, rows_v1, rows_v2)
    out_bufs = (out_v0, out_v1)
    gsems = (gsem0, gsem1, gsem2)
    wsems = (wsem0, wsem1)

    pltpu.sync_copy(idx_hbm.at[pl.ds(base_q * K, _QPW * K)], idx_all)

    def start_gather(ch):
        p = ch % _GDEPTH
        isl = idx_all.at[pl.ds(ch * _QCHUNK * K, _QCHUNK * K)]
        return pltpu.async_copy(table_hbm.at[isl], row_bufs[p], gsems[p])

    gcop = [start_gather(ch) for ch in range(_GDEPTH)]
    wcop = [None, None]
    for ch in range(_NCHUNK):
        gcop[ch].wait()
        rows_v = row_bufs[ch % _GDEPTH]
        out_v = out_bufs[ch % 2]
        if wcop[ch % 2] is not None:
            wcop[ch % 2].wait()

        def q_body(q, carry2):
            for l in range(C // 16):
                sl = pl.ds(l * 16, 16)
                acc = (rows_v[K * q, sl] + rows_v[K * q + 1, sl]
                       + rows_v[K * q + 2, sl] + rows_v[K * q + 3, sl])
                out_v[q, sl] = acc * jnp.float32(1.0 / K)
            return carry2

        lax.fori_loop(0, _QCHUNK, q_body, 0, unroll=4)
        if ch + _GDEPTH < _NCHUNK:
            gcop.append(start_gather(ch + _GDEPTH))
        qb = base_q + ch * _QCHUNK
        wcop[ch % 2] = pltpu.async_copy(
            out_v, out_hbm.at[pl.ds(qb, _QCHUNK)], wsems[ch % 2])
    wcop[0].wait()
    wcop[1].wait()


def _sc_gather_mean(table, idx_flat):
    mesh = plsc.VectorSubcoreMesh(core_axis_name="c", subcore_axis_name="s")
    kern = pl.kernel(
        _gather_mean_body,
        out_type=jax.ShapeDtypeStruct((N_TOTAL, C), jnp.float32),
        mesh=mesh,
        scratch_types=[
            pltpu.VMEM((_QPW * K,), jnp.int32),
            pltpu.VMEM((_QCHUNK * K, C), jnp.float32),
            pltpu.VMEM((_QCHUNK * K, C), jnp.float32),
            pltpu.VMEM((_QCHUNK * K, C), jnp.float32),
            pltpu.VMEM((_QCHUNK, C), jnp.float32),
            pltpu.VMEM((_QCHUNK, C), jnp.float32),
            pltpu.SemaphoreType.DMA,
            pltpu.SemaphoreType.DMA,
            pltpu.SemaphoreType.DMA,
            pltpu.SemaphoreType.DMA,
            pltpu.SemaphoreType.DMA,
        ],
    )
    return kern(table, idx_flat)


def kernel(values, coords, new_coords, shift):
    all_coords = jnp.concatenate([coords, new_coords], axis=0)
    kx = coords[:, 0][None, :]
    ky = coords[:, 1][None, :]
    idx = _tc_topk(all_coords, shift[None, :], kx, ky)
    table = values.T
    out_rows = _sc_gather_mean(table, idx.reshape(-1))
    return out_rows.T


# speedup vs baseline: 1.5455x; 1.0243x over previous
import functools

import jax
import jax.numpy as jnp
from jax import lax
from jax.experimental import pallas as pl
from jax.experimental.pallas import tpu as pltpu
from jax.experimental.pallas import tpu_sc as plsc

N_IN = 2048
N_TOTAL = 8192
C = 128
K = 4


_QB = 512


def _topk_body(q_ref, sh_ref, kx_ref, ky_ref, idx_ref):
    qx = q_ref[:, 0:1] - sh_ref[0:1, 0:1]
    qy = q_ref[:, 1:2] - sh_ref[0:1, 1:2]
    dx = qx - kx_ref[...]
    dy = qy - ky_ref[...]
    d = jnp.sqrt(dx * dx + dy * dy)
    iota_f = lax.broadcasted_iota(jnp.int32, (_QB, N_IN), 1).astype(jnp.float32)
    cols = []
    for _ in range(K):
        m = jnp.min(d, axis=1, keepdims=True)
        j = jnp.min(jnp.where(d == m, iota_f, jnp.float32(N_IN)),
                    axis=1, keepdims=True)
        d = jnp.where(iota_f == j, jnp.float32(jnp.inf), d)
        cols.append(j)
    idx_ref[...] = jnp.concatenate(cols, axis=1).astype(jnp.int32)


def _tc_topk(all_coords, shift2d, kx, ky):
    grid = all_coords.shape[0] // _QB
    return pl.pallas_call(
        _topk_body,
        grid=(grid,),
        in_specs=[
            pl.BlockSpec((_QB, 2), lambda i: (i, 0)),
            pl.BlockSpec((1, 2), lambda i: (0, 0)),
            pl.BlockSpec((1, N_IN), lambda i: (0, 0)),
            pl.BlockSpec((1, N_IN), lambda i: (0, 0)),
        ],
        out_specs=pl.BlockSpec((_QB, K), lambda i: (i, 0)),
        out_shape=jax.ShapeDtypeStruct((all_coords.shape[0], K), jnp.int32),
    )(all_coords, shift2d, kx, ky)


_NC = 2
_NS = 16
_NW = _NC * _NS
_QPW = N_TOTAL // (2 * _NW)
_QCHUNK = 32
_NCHUNK = _QPW // _QCHUNK


_GDEPTH = 3


def _gather_mean_body(table_hbm, idx_hbm, out_hbm,
                      idx_all, rows_v0, rows_v1, rows_v2,
                      out_v0, out_v1,
                      gsem0, gsem1, gsem2, wsem0, wsem1):
    c = lax.axis_index("c")
    s = lax.axis_index("s")
    wid = s * _NC + c
    base_q = wid * _QPW
    row_bufs = (rows_v0, rows_v1, rows_v2)
    out_bufs = (out_v0, out_v1)
    gsems = (gsem0, gsem1, gsem2)
    wsems = (wsem0, wsem1)

    pltpu.sync_copy(idx_hbm.at[pl.ds(base_q * K, _QPW * K)], idx_all)

    def start_gather(ch):
        p = ch % _GDEPTH
        isl = idx_all.at[pl.ds(ch * _QCHUNK * K, _QCHUNK * K)]
        return pltpu.async_copy(table_hbm.at[isl], row_bufs[p], gsems[p])

    gcop = [start_gather(ch) for ch in range(_GDEPTH)]
    wcop = [None, None]
    for ch in range(_NCHUNK):
        gcop[ch].wait()
        rows_v = row_bufs[ch % _GDEPTH]
        out_v = out_bufs[ch % 2]
        if wcop[ch % 2] is not None:
            wcop[ch % 2].wait()

        def q_body(q, carry2):
            for l in range(C // 16):
                sl = pl.ds(l * 16, 16)
                acc = (rows_v[K * q, sl] + rows_v[K * q + 1, sl]
                       + rows_v[K * q + 2, sl] + rows_v[K * q + 3, sl])
                out_v[q, sl] = acc * jnp.float32(1.0 / K)
            return carry2

        lax.fori_loop(0, _QCHUNK, q_body, 0, unroll=4)
        if ch + _GDEPTH < _NCHUNK:
            gcop.append(start_gather(ch + _GDEPTH))
        qb = base_q + ch * _QCHUNK
        wcop[ch % 2] = pltpu.async_copy(
            out_v, out_hbm.at[pl.ds(qb, _QCHUNK)], wsems[ch % 2])
    wcop[0].wait()
    wcop[1].wait()


def _sc_gather_mean(table, idx_flat):
    n_q = idx_flat.shape[0] // K
    mesh = plsc.VectorSubcoreMesh(core_axis_name="c", subcore_axis_name="s")
    kern = pl.kernel(
        _gather_mean_body,
        out_type=jax.ShapeDtypeStruct((n_q, C), jnp.float32),
        mesh=mesh,
        scratch_types=[
            pltpu.VMEM((_QPW * K,), jnp.int32),
            pltpu.VMEM((_QCHUNK * K, C), jnp.float32),
            pltpu.VMEM((_QCHUNK * K, C), jnp.float32),
            pltpu.VMEM((_QCHUNK * K, C), jnp.float32),
            pltpu.VMEM((_QCHUNK, C), jnp.float32),
            pltpu.VMEM((_QCHUNK, C), jnp.float32),
            pltpu.SemaphoreType.DMA,
            pltpu.SemaphoreType.DMA,
            pltpu.SemaphoreType.DMA,
            pltpu.SemaphoreType.DMA,
            pltpu.SemaphoreType.DMA,
        ],
    )
    return kern(table, idx_flat)


def kernel(values, coords, new_coords, shift):
    all_coords = jnp.concatenate([coords, new_coords], axis=0)
    kx = coords[:, 0][None, :]
    ky = coords[:, 1][None, :]
    table = values.T
    half = N_TOTAL // 2
    sh2 = shift[None, :]
    idx0 = _tc_topk(all_coords[:half], sh2, kx, ky)
    idx1 = _tc_topk(all_coords[half:], sh2, kx, ky)
    out0 = _sc_gather_mean(table, idx0.reshape(-1))
    out1 = _sc_gather_mean(table, idx1.reshape(-1))
    return jnp.concatenate([out0, out1], axis=0).T
